# baseline (speedup 1.0000x reference)
import functools

import jax
import jax.numpy as jnp
from jax.experimental import pallas as pl
from jax.experimental.pallas import tpu as pltpu


def _pick_bm(n):
    for bm in (200, 512, 400, 256, 128, 80, 64, 40, 32, 16, 8):
        if n % bm == 0:
            return bm
    return n


def _body(adj_ref, feat_ref, w_ref, out_ref, h_ref, *, bm):
    i = pl.program_id(0)

    @pl.when(i == 0)
    def _():
        h_ref[...] = 0.5 * jnp.dot(
            feat_ref[...], w_ref[...], preferred_element_type=jnp.float32)

    h_rows = h_ref[pl.ds(i * bm, bm), :]
    out_ref[...] = h_rows + jnp.dot(
        adj_ref[...], h_ref[...], preferred_element_type=jnp.float32)


@jax.jit
def kernel(feature, adj, W):
    n, d = feature.shape
    e = W.shape[1]
    bm = _pick_bm(n)
    grid = n // bm
    return pl.pallas_call(
        functools.partial(_body, bm=bm),
        grid=(grid,),
        in_specs=[
            pl.BlockSpec((bm, n), lambda i: (i, 0)),
            pl.BlockSpec((n, d), lambda i: (0, 0)),
            pl.BlockSpec((d, e), lambda i: (0, 0)),
        ],
        out_specs=pl.BlockSpec((bm, e), lambda i: (i, 0)),
        out_shape=jax.ShapeDtypeStruct((n, e), jnp.float32),
        scratch_shapes=[pltpu.VMEM((n, e), jnp.float32)],
        compiler_params=pltpu.CompilerParams(
            dimension_semantics=("arbitrary",),
        ),
    )(adj, feature, W)


# speedup vs baseline: 1.0065x; 1.0065x over previous
import functools

import jax
import jax.numpy as jnp
from jax.experimental import pallas as pl
from jax.experimental.pallas import tpu as pltpu


def _pick_bm(n):
    for bm in (512, 400, 256, 200, 128, 80, 64, 40, 32, 16, 8):
        if n % bm == 0:
            return bm
    return n


def _body(adj_ref, feat_ref, w_ref, out_ref, h_ref, *, bm):
    i = pl.program_id(0)

    @pl.when(i == 0)
    def _():
        h_ref[...] = 0.5 * jnp.dot(
            feat_ref[...], w_ref[...], preferred_element_type=jnp.float32)

    h_rows = h_ref[pl.ds(i * bm, bm), :]
    out_ref[...] = h_rows + jnp.dot(
        adj_ref[...], h_ref[...], preferred_element_type=jnp.float32)


@jax.jit
def kernel(feature, adj, W):
    n, d = feature.shape
    e = W.shape[1]
    bm = _pick_bm(n)
    grid = n // bm
    return pl.pallas_call(
        functools.partial(_body, bm=bm),
        grid=(grid,),
        in_specs=[
            pl.BlockSpec((bm, n), lambda i: (i, 0)),
            pl.BlockSpec((n, d), lambda i: (0, 0)),
            pl.BlockSpec((d, e), lambda i: (0, 0)),
        ],
        out_specs=pl.BlockSpec((bm, e), lambda i: (i, 0)),
        out_shape=jax.ShapeDtypeStruct((n, e), jnp.float32),
        scratch_shapes=[pltpu.VMEM((n, e), jnp.float32)],
        compiler_params=pltpu.CompilerParams(
            dimension_semantics=("arbitrary",),
        ),
    )(adj, feature, W)
